# baseline (speedup 1.0000x reference)
import jax
import jax.numpy as jnp
from jax.experimental import pallas as pl

TEMP = 8.0


HC = 4


def _attn_kernel(qt_ref, kt_ref, vt_ref, m_ref, ot_ref):
    keep = m_ref[0] != 0
    for h in range(HC):
        qt = qt_ref[0, h] * (1.0 / TEMP)
        kt = kt_ref[0, h]
        vt = vt_ref[0, h]
        s = jax.lax.dot_general(
            qt, kt, (((0,), (0,)), ((), ())),
            preferred_element_type=jnp.float32)
        sm = jnp.where(keep, s, -jnp.inf)
        mx = jnp.max(sm, axis=-1, keepdims=True)
        mx = jnp.where(jnp.isfinite(mx), mx, 0.0)
        ex = jnp.exp(sm - mx)
        den = jnp.sum(ex, axis=-1, keepdims=True)
        r = jnp.where(den == 0.0, 0.0, 1.0 / den)
        o = jax.lax.dot_general(
            vt, ex, (((1,), (1,)), ((), ())),
            preferred_element_type=jnp.float32)
        ot_ref[0, h] = o * r.reshape(1, -1)


def kernel(q, k, v, mask):
    B, H, L, d = q.shape
    qt = jnp.transpose(q, (0, 1, 3, 2))
    kt = jnp.transpose(k, (0, 1, 3, 2))
    vt = jnp.transpose(v, (0, 1, 3, 2))
    m8 = mask.view(jnp.int8)
    nh = H // HC
    ot = pl.pallas_call(
        _attn_kernel,
        grid=(B * nh,),
        in_specs=[
            pl.BlockSpec((1, HC, d, L), lambda i: (i // nh, i % nh, 0, 0)),
            pl.BlockSpec((1, HC, d, L), lambda i: (i // nh, i % nh, 0, 0)),
            pl.BlockSpec((1, HC, d, L), lambda i: (i // nh, i % nh, 0, 0)),
            pl.BlockSpec((1, L, L), lambda i: (i // nh, 0, 0)),
        ],
        out_specs=pl.BlockSpec((1, HC, d, L), lambda i: (i // nh, i % nh, 0, 0)),
        out_shape=jax.ShapeDtypeStruct((B, H, d, L), jnp.float32),
    )(qt, kt, vt, m8)
    return jnp.transpose(ot, (0, 1, 3, 2))


# speedup vs baseline: 1.0339x; 1.0339x over previous
import jax
import jax.numpy as jnp
from jax.experimental import pallas as pl

TEMP = 8.0


HC = 8


BIG = 1e30


def _attn_kernel(qt_ref, kt_ref, vt_ref, m_ref, ot_ref):
    negm = (m_ref[0].astype(jnp.float32) - 1.0) * BIG
    rowvalid = jnp.max(negm, axis=-1, keepdims=True) > -1.0
    for h in range(HC):
        qt = qt_ref[0, h] * (1.0 / TEMP)
        kt = kt_ref[0, h]
        vt = vt_ref[0, h]
        s = jax.lax.dot_general(
            qt, kt, (((0,), (0,)), ((), ())),
            preferred_element_type=jnp.float32)
        sm = s + negm
        mx = jnp.max(sm, axis=-1, keepdims=True)
        ex = jnp.exp(sm - mx)
        den = jnp.sum(ex, axis=-1, keepdims=True)
        r = jnp.where(rowvalid, 1.0 / den, 0.0)
        o = jax.lax.dot_general(
            vt, ex, (((1,), (1,)), ((), ())),
            preferred_element_type=jnp.float32)
        ot_ref[0, h] = o * r.reshape(1, -1)


def kernel(q, k, v, mask):
    B, H, L, d = q.shape
    qt = jnp.transpose(q, (0, 1, 3, 2))
    kt = jnp.transpose(k, (0, 1, 3, 2))
    vt = jnp.transpose(v, (0, 1, 3, 2))
    m8 = mask.view(jnp.int8)
    nh = H // HC
    ot = pl.pallas_call(
        _attn_kernel,
        grid=(B * nh,),
        in_specs=[
            pl.BlockSpec((1, HC, d, L), lambda i: (i // nh, i % nh, 0, 0)),
            pl.BlockSpec((1, HC, d, L), lambda i: (i // nh, i % nh, 0, 0)),
            pl.BlockSpec((1, HC, d, L), lambda i: (i // nh, i % nh, 0, 0)),
            pl.BlockSpec((1, L, L), lambda i: (i // nh, 0, 0)),
        ],
        out_specs=pl.BlockSpec((1, HC, d, L), lambda i: (i // nh, i % nh, 0, 0)),
        out_shape=jax.ShapeDtypeStruct((B, H, d, L), jnp.float32),
    )(qt, kt, vt, m8)
    return jnp.transpose(ot, (0, 1, 3, 2))


# speedup vs baseline: 1.0406x; 1.0065x over previous
import jax
import jax.numpy as jnp
from jax.experimental import pallas as pl

TEMP = 8.0


HC = 4


BIG = 1e30


def _attn_kernel(qt_ref, kt_ref, vt_ref, m_ref, ot_ref):
    negm = (m_ref[0].astype(jnp.float32) - 1.0) * BIG
    rowvalid = jnp.max(negm, axis=-1, keepdims=True) > -1.0
    for h in range(HC):
        qt = qt_ref[0, h] * (1.0 / TEMP)
        kt = kt_ref[0, h]
        vt = vt_ref[0, h]
        s = jax.lax.dot_general(
            qt, kt, (((0,), (0,)), ((), ())),
            preferred_element_type=jnp.float32)
        sm = s + negm
        mx = jnp.max(sm, axis=-1, keepdims=True)
        ex = jnp.exp(sm - mx)
        den = jnp.sum(ex, axis=-1, keepdims=True)
        r = jnp.where(rowvalid, 1.0 / den, 0.0)
        o = jax.lax.dot_general(
            vt, ex, (((1,), (1,)), ((), ())),
            preferred_element_type=jnp.float32)
        ot_ref[0, h] = o * r.reshape(1, -1)


def kernel(q, k, v, mask):
    B, H, L, d = q.shape
    qt = jnp.transpose(q, (0, 1, 3, 2))
    kt = jnp.transpose(k, (0, 1, 3, 2))
    vt = jnp.transpose(v, (0, 1, 3, 2))
    m8 = mask.view(jnp.int8)
    nh = H // HC
    ot = pl.pallas_call(
        _attn_kernel,
        grid=(B * nh,),
        in_specs=[
            pl.BlockSpec((1, HC, d, L), lambda i: (i // nh, i % nh, 0, 0)),
            pl.BlockSpec((1, HC, d, L), lambda i: (i // nh, i % nh, 0, 0)),
            pl.BlockSpec((1, HC, d, L), lambda i: (i // nh, i % nh, 0, 0)),
            pl.BlockSpec((1, L, L), lambda i: (i // nh, 0, 0)),
        ],
        out_specs=pl.BlockSpec((1, HC, d, L), lambda i: (i // nh, i % nh, 0, 0)),
        out_shape=jax.ShapeDtypeStruct((B, H, d, L), jnp.float32),
    )(qt, kt, vt, m8)
    return jnp.transpose(ot, (0, 1, 3, 2))
